# baseline (speedup 1.0000x reference)
import functools

import jax
import jax.numpy as jnp
from jax import lax
from jax.experimental import pallas as pl
from jax.experimental.pallas import tpu as pltpu
from jax.experimental.pallas import tpu_sc as plsc

_MAX_LEN = 8192
_CHUNK = 16
_NBUF = 4
_OG = 2


@functools.lru_cache(maxsize=None)
def _make_gather(B, V, D):
    info = plsc.get_sparse_core_info()
    nc, ns = info.num_cores, info.num_subcores
    nw = nc * ns
    b_per_w = B // nw
    n_chunks = b_per_w // _CHUNK
    assert b_per_w * nw == B and n_chunks * _CHUNK == b_per_w
    assert n_chunks % _NBUF == 0 and n_chunks >= 2 * _NBUF

    mesh = plsc.VectorSubcoreMesh(core_axis_name="c", subcore_axis_name="s")

    @functools.partial(
        pl.kernel,
        mesh=mesh,
        out_type=jax.ShapeDtypeStruct((B, D), jnp.float32),
        scratch_types=[
            pltpu.VMEM((b_per_w,), jnp.int32),
            pltpu.VMEM((_NBUF, _CHUNK, D), jnp.float32),
        ]
        + [pltpu.SemaphoreType.DMA] * (2 * _NBUF),
    )
    def gather_kernel(idx_hbm, table_hbm, out_hbm, idx_v, rows_v, *sems):
        gsem, wsem = sems[:_NBUF], sems[_NBUF:]
        wid = lax.axis_index("s") * nc + lax.axis_index("c")
        base = wid * b_per_w
        pltpu.sync_copy(idx_hbm.at[pl.ds(base, b_per_w)], idx_v)

        def clamp_body(i, carry):
            off = pl.multiple_of(i * 16, 16)
            v = idx_v[pl.ds(off, 16)]
            idx_v[pl.ds(off, 16)] = jnp.clip(v, 0, V - 1)
            return carry

        lax.fori_loop(0, b_per_w // 16, clamp_body, 0)

        def start(chunk, buf):
            off = pl.multiple_of(chunk * _CHUNK, _CHUNK)
            pltpu.async_copy(
                table_hbm.at[idx_v.at[pl.ds(off, _CHUNK)]],
                rows_v.at[buf],
                gsem[buf],
            )

        def wait_gather(buf):
            pltpu.make_async_copy(
                table_hbm.at[idx_v.at[pl.ds(0, _CHUNK)]],
                rows_v.at[buf],
                gsem[buf],
            ).wait()

        def out_slice(chunk):
            return out_hbm.at[pl.ds(pl.multiple_of(base + chunk * _CHUNK, _CHUNK), _CHUNK)]

        def start_write(chunk, buf):
            pltpu.async_copy(rows_v.at[buf], out_slice(chunk), wsem[buf])

        def wait_write(buf):
            pltpu.make_async_copy(rows_v.at[buf], out_slice(0), wsem[buf]).wait()

        for b in range(_OG):
            start(b, b)

        def body(g, carry):
            for b in range(_NBUF):
                chunk = _NBUF * g + b
                nxt = chunk + _OG
                bn = (b + _OG) % _NBUF

                @pl.when(nxt < n_chunks)
                def _():
                    @pl.when(nxt >= _NBUF)
                    def _():
                        wait_write(bn)

                    start(nxt, bn)

                wait_gather(b)
                start_write(chunk, b)
            return carry

        lax.fori_loop(0, n_chunks // _NBUF, body, 0)
        for b in range(_NBUF):
            wait_write(b)

    return gather_kernel


def kernel(positions, table):
    out_shape = positions.shape + (table.shape[1],)
    flat = positions.reshape(-1)
    out = _make_gather(flat.shape[0], table.shape[0], table.shape[1])(flat, table)
    return out.reshape(out_shape)


# speedup vs baseline: 1.0001x; 1.0001x over previous
import functools

import jax
import jax.numpy as jnp
from jax import lax
from jax.experimental import pallas as pl
from jax.experimental.pallas import tpu as pltpu
from jax.experimental.pallas import tpu_sc as plsc

_CHUNK = 16
_NBUF = 4
_OG = 2


@functools.lru_cache(maxsize=None)
def _make_gather(BATCH, SEQ, V, D):
    info = plsc.get_sparse_core_info()
    nc, ns = info.num_cores, info.num_subcores
    nw = nc * ns
    B = BATCH * SEQ
    b_per_w = B // nw
    n_chunks = b_per_w // _CHUNK
    assert b_per_w * nw == B and n_chunks * _CHUNK == b_per_w
    assert n_chunks % _NBUF == 0 and n_chunks >= 2 * _NBUF
    assert SEQ % b_per_w == 0

    mesh = plsc.VectorSubcoreMesh(core_axis_name="c", subcore_axis_name="s")

    @functools.partial(
        pl.kernel,
        mesh=mesh,
        out_type=jax.ShapeDtypeStruct((BATCH, SEQ, D), jnp.float32),
        scratch_types=[
            pltpu.VMEM((b_per_w,), jnp.int32),
            pltpu.VMEM((_NBUF, _CHUNK, D), jnp.float32),
        ]
        + [pltpu.SemaphoreType.DMA] * (2 * _NBUF),
    )
    def gather_kernel(pos_hbm, table_hbm, out_hbm, idx_v, rows_v, *sems):
        gsem, wsem = sems[:_NBUF], sems[_NBUF:]
        wid = lax.axis_index("s") * nc + lax.axis_index("c")
        bidx = wid // (SEQ // b_per_w)
        seq_base = (wid % (SEQ // b_per_w)) * b_per_w
        pltpu.sync_copy(pos_hbm.at[bidx, pl.ds(seq_base, b_per_w)], idx_v)

        def clamp_body(i, carry):
            off = pl.multiple_of(i * 16, 16)
            v = idx_v[pl.ds(off, 16)]
            idx_v[pl.ds(off, 16)] = jnp.clip(v, 0, V - 1)
            return carry

        lax.fori_loop(0, b_per_w // 16, clamp_body, 0)

        def start(chunk, buf):
            off = pl.multiple_of(chunk * _CHUNK, _CHUNK)
            pltpu.async_copy(
                table_hbm.at[idx_v.at[pl.ds(off, _CHUNK)]],
                rows_v.at[buf],
                gsem[buf],
            )

        def wait_gather(buf):
            pltpu.make_async_copy(
                table_hbm.at[idx_v.at[pl.ds(0, _CHUNK)]],
                rows_v.at[buf],
                gsem[buf],
            ).wait()

        def out_slice(chunk):
            off = pl.multiple_of(seq_base + chunk * _CHUNK, _CHUNK)
            return out_hbm.at[bidx, pl.ds(off, _CHUNK)]

        def start_write(chunk, buf):
            pltpu.async_copy(rows_v.at[buf], out_slice(chunk), wsem[buf])

        def wait_write(buf):
            pltpu.make_async_copy(rows_v.at[buf], out_slice(0), wsem[buf]).wait()

        for b in range(_OG):
            start(b, b)

        def body(g, carry):
            for b in range(_NBUF):
                chunk = _NBUF * g + b
                nxt = chunk + _OG
                bn = (b + _OG) % _NBUF

                @pl.when(nxt < n_chunks)
                def _():
                    @pl.when(nxt >= _NBUF)
                    def _():
                        wait_write(bn)

                    start(nxt, bn)

                wait_gather(b)
                start_write(chunk, b)
            return carry

        lax.fori_loop(0, n_chunks // _NBUF, body, 0)
        for b in range(_NBUF):
            wait_write(b)

    return gather_kernel


def kernel(positions, table):
    return _make_gather(
        positions.shape[0], positions.shape[1], table.shape[0], table.shape[1]
    )(positions, table)


# speedup vs baseline: 1.0098x; 1.0097x over previous
import functools

import jax
import jax.numpy as jnp
from jax import lax
from jax.experimental import pallas as pl
from jax.experimental.pallas import tpu as pltpu
from jax.experimental.pallas import tpu_sc as plsc

_CHUNK = 16
_NBUF = 4
_OG = 2


@functools.lru_cache(maxsize=None)
def _make_gather(BATCH, SEQ, V, D):
    info = plsc.get_sparse_core_info()
    nc, ns = info.num_cores, info.num_subcores
    nw = nc * ns
    B = BATCH * SEQ
    b_per_w = B // nw
    n_chunks = b_per_w // _CHUNK
    assert b_per_w * nw == B and n_chunks * _CHUNK == b_per_w
    assert n_chunks % 8 == 0 and n_chunks >= 16
    assert SEQ % b_per_w == 0

    mesh = plsc.VectorSubcoreMesh(core_axis_name="c", subcore_axis_name="s")

    @functools.partial(
        pl.kernel,
        mesh=mesh,
        out_type=jax.ShapeDtypeStruct((BATCH, SEQ, D), jnp.float32),
        scratch_types=[
            pltpu.VMEM((b_per_w,), jnp.int32),
            pltpu.VMEM((_NBUF, _CHUNK, D), jnp.float32),
            pltpu.VMEM_SHARED((ns, 2, _CHUNK, D), jnp.float32),
        ]
        + [pltpu.SemaphoreType.DMA] * (_NBUF + 2 + 2 + 2),
    )
    def gather_kernel(pos_hbm, table_hbm, out_hbm, idx_v, rows_v, stage_v, *sems):
        gsem = sems[:_NBUF]
        wsem = sems[_NBUF:_NBUF + 2]
        h1sem = sems[_NBUF + 2:_NBUF + 4]
        h2sem = sems[_NBUF + 4:]
        sid = lax.axis_index("s")
        wid = sid * nc + lax.axis_index("c")
        bidx = wid // (SEQ // b_per_w)
        seq_base = (wid % (SEQ // b_per_w)) * b_per_w
        pltpu.sync_copy(pos_hbm.at[bidx, pl.ds(seq_base, b_per_w)], idx_v)

        def clamp_body(i, carry):
            off = pl.multiple_of(i * 16, 16)
            idx_v[pl.ds(off, 16)] = jnp.clip(idx_v[pl.ds(off, 16)], 0, V - 1)
            return carry

        lax.fori_loop(0, b_per_w // 16, clamp_body, 0)

        def start(chunk, buf):
            off = pl.multiple_of(chunk * _CHUNK, _CHUNK)
            pltpu.async_copy(
                table_hbm.at[idx_v.at[pl.ds(off, _CHUNK)]], rows_v.at[buf], gsem[buf]
            )

        def wait_gather(buf):
            pltpu.make_async_copy(
                table_hbm.at[idx_v.at[pl.ds(0, _CHUNK)]], rows_v.at[buf], gsem[buf]
            ).wait()

        def out_slice(chunk):
            off = pl.multiple_of(seq_base + chunk * _CHUNK, _CHUNK)
            return out_hbm.at[bidx, pl.ds(off, _CHUNK)]

        def slot_ref(s):
            return stage_v.at[sid, s]

        for b in range(_OG):
            start(b, b)

        def body8(g, carry):
            for k in range(8):
                chunk = 8 * g + k
                b = k % _NBUF
                nxt = chunk + _OG
                bn = (b + _OG) % _NBUF

                @pl.when(nxt < n_chunks)
                def _():
                    @pl.when(nxt >= _NBUF)
                    def _():
                        if bn % 2 == 0:
                            pltpu.make_async_copy(
                                rows_v.at[bn], out_slice(0), wsem[bn // 2]
                            ).wait()
                        else:
                            bnp = (bn - 1) // 2
                            pltpu.make_async_copy(
                                rows_v.at[bn], slot_ref(0), h1sem[bnp]
                            ).wait()
                            pltpu.async_copy(
                                slot_ref(bnp), out_slice(nxt - _NBUF), h2sem[bnp]
                            )

                    start(nxt, bn)

                wait_gather(b)
                if b % 2 == 0:
                    pltpu.async_copy(rows_v.at[b], out_slice(chunk), wsem[b // 2])
                else:
                    bp = (b - 1) // 2

                    @pl.when(chunk >= _NBUF + 1)
                    def _():
                        pltpu.make_async_copy(
                            slot_ref(bp), out_slice(0), h2sem[bp]
                        ).wait()

                    pltpu.async_copy(rows_v.at[b], slot_ref(bp), h1sem[bp])
            return carry

        lax.fori_loop(0, n_chunks // 8, body8, 0)

        for bp, last in ((0, n_chunks - 3), (1, n_chunks - 1)):
            pltpu.make_async_copy(rows_v.at[2 * bp + 1], slot_ref(0), h1sem[bp]).wait()
            pltpu.async_copy(slot_ref(bp), out_slice(last), h2sem[bp])
        for s in range(2):
            pltpu.make_async_copy(slot_ref(s), out_slice(0), h2sem[s]).wait()
        for w in range(2):
            pltpu.make_async_copy(rows_v.at[2 * w], out_slice(0), wsem[w]).wait()

    return gather_kernel


def kernel(positions, table):
    return _make_gather(
        positions.shape[0], positions.shape[1], table.shape[0], table.shape[1]
    )(positions, table)


# speedup vs baseline: 1.0140x; 1.0041x over previous
import functools

import jax
import jax.numpy as jnp
from jax import lax
from jax.experimental import pallas as pl
from jax.experimental.pallas import tpu as pltpu
from jax.experimental.pallas import tpu_sc as plsc

_CHUNK = 16
_NBUF = 4
_OG = 2


@functools.lru_cache(maxsize=None)
def _make_gather(BATCH, SEQ, V, D):
    info = plsc.get_sparse_core_info()
    nc, ns = info.num_cores, info.num_subcores
    nw = nc * ns
    B = BATCH * SEQ
    b_per_w = B // nw
    n_chunks = b_per_w // _CHUNK
    assert b_per_w * nw == B and n_chunks * _CHUNK == b_per_w
    assert n_chunks % 8 == 0 and n_chunks >= 16
    assert SEQ % b_per_w == 0

    mesh = plsc.VectorSubcoreMesh(core_axis_name="c", subcore_axis_name="s")

    @functools.partial(
        pl.kernel,
        mesh=mesh,
        out_type=jax.ShapeDtypeStruct((BATCH, SEQ, D), jnp.float32),
        scratch_types=[
            pltpu.VMEM((b_per_w,), jnp.int32),
            pltpu.VMEM((_NBUF, _CHUNK, D), jnp.float32),
            pltpu.VMEM_SHARED((ns, 3, _CHUNK, D), jnp.float32),
        ]
        + [pltpu.SemaphoreType.DMA] * (_NBUF + 1 + 3 + 3),
    )
    def gather_kernel(pos_hbm, table_hbm, out_hbm, idx_v, rows_v, stage_v, *sems):
        gsem = sems[:_NBUF]
        wsem = sems[_NBUF:_NBUF + 1]
        h1sem = sems[_NBUF + 1:_NBUF + 4]
        h2sem = sems[_NBUF + 4:]
        sid = lax.axis_index("s")
        wid = sid * nc + lax.axis_index("c")
        bidx = wid // (SEQ // b_per_w)
        seq_base = (wid % (SEQ // b_per_w)) * b_per_w
        pltpu.sync_copy(pos_hbm.at[bidx, pl.ds(seq_base, b_per_w)], idx_v)

        def clamp_body(i, carry):
            off = pl.multiple_of(i * 16, 16)
            idx_v[pl.ds(off, 16)] = jnp.clip(idx_v[pl.ds(off, 16)], 0, V - 1)
            return carry

        lax.fori_loop(0, b_per_w // 16, clamp_body, 0)

        def start(chunk, buf):
            off = pl.multiple_of(chunk * _CHUNK, _CHUNK)
            pltpu.async_copy(
                table_hbm.at[idx_v.at[pl.ds(off, _CHUNK)]], rows_v.at[buf], gsem[buf]
            )

        def wait_gather(buf):
            pltpu.make_async_copy(
                table_hbm.at[idx_v.at[pl.ds(0, _CHUNK)]], rows_v.at[buf], gsem[buf]
            ).wait()

        def out_slice(chunk):
            off = pl.multiple_of(seq_base + chunk * _CHUNK, _CHUNK)
            return out_hbm.at[bidx, pl.ds(off, _CHUNK)]

        def slot_ref(s):
            return stage_v.at[sid, s]

        for b in range(_OG):
            start(b, b)

        def body8(g, carry):
            for k in range(8):
                chunk = 8 * g + k
                b = k % _NBUF
                nxt = chunk + _OG
                bn = (b + _OG) % _NBUF

                @pl.when(nxt < n_chunks)
                def _():
                    @pl.when(nxt >= _NBUF)
                    def _():
                        if bn == 0:
                            pltpu.make_async_copy(
                                rows_v.at[bn], out_slice(0), wsem[0]
                            ).wait()
                        else:
                            bnp = bn - 1
                            pltpu.make_async_copy(
                                rows_v.at[bn], slot_ref(0), h1sem[bnp]
                            ).wait()
                            pltpu.async_copy(
                                slot_ref(bnp), out_slice(nxt - _NBUF), h2sem[bnp]
                            )

                    start(nxt, bn)

                wait_gather(b)
                if b == 0:
                    pltpu.async_copy(rows_v.at[b], out_slice(chunk), wsem[0])
                else:
                    bp = b - 1

                    @pl.when(chunk >= _NBUF + 1)
                    def _():
                        pltpu.make_async_copy(
                            slot_ref(bp), out_slice(0), h2sem[bp]
                        ).wait()

                    pltpu.async_copy(rows_v.at[b], slot_ref(bp), h1sem[bp])
            return carry

        lax.fori_loop(0, n_chunks // 8, body8, 0)

        for bp, last in ((0, n_chunks - 3), (1, n_chunks - 2), (2, n_chunks - 1)):
            pltpu.make_async_copy(rows_v.at[bp + 1], slot_ref(0), h1sem[bp]).wait()
            pltpu.async_copy(slot_ref(bp), out_slice(last), h2sem[bp])
        for s in range(3):
            pltpu.make_async_copy(slot_ref(s), out_slice(0), h2sem[s]).wait()
        pltpu.make_async_copy(rows_v.at[0], out_slice(0), wsem[0]).wait()

    return gather_kernel


def kernel(positions, table):
    return _make_gather(
        positions.shape[0], positions.shape[1], table.shape[0], table.shape[1]
    )(positions, table)


# speedup vs baseline: 1.0142x; 1.0002x over previous
import functools

import jax
import jax.numpy as jnp
from jax import lax
from jax.experimental import pallas as pl
from jax.experimental.pallas import tpu as pltpu
from jax.experimental.pallas import tpu_sc as plsc

_CHUNK = 16
_NBUF = 4
_OG = 3


@functools.lru_cache(maxsize=None)
def _make_gather(BATCH, SEQ, V, D):
    info = plsc.get_sparse_core_info()
    nc, ns = info.num_cores, info.num_subcores
    nw = nc * ns
    B = BATCH * SEQ
    b_per_w = B // nw
    n_chunks = b_per_w // _CHUNK
    assert b_per_w * nw == B and n_chunks * _CHUNK == b_per_w
    assert n_chunks % 8 == 0 and n_chunks >= 16
    assert SEQ % b_per_w == 0

    mesh = plsc.VectorSubcoreMesh(core_axis_name="c", subcore_axis_name="s")

    @functools.partial(
        pl.kernel,
        mesh=mesh,
        out_type=jax.ShapeDtypeStruct((BATCH, SEQ, D), jnp.float32),
        scratch_types=[
            pltpu.VMEM((b_per_w,), jnp.int32),
            pltpu.VMEM((_NBUF, _CHUNK, D), jnp.float32),
            pltpu.VMEM_SHARED((ns, 3, _CHUNK, D), jnp.float32),
        ]
        + [pltpu.SemaphoreType.DMA] * (_NBUF + 1 + 3 + 3),
    )
    def gather_kernel(pos_hbm, table_hbm, out_hbm, idx_v, rows_v, stage_v, *sems):
        gsem = sems[:_NBUF]
        wsem = sems[_NBUF:_NBUF + 1]
        h1sem = sems[_NBUF + 1:_NBUF + 4]
        h2sem = sems[_NBUF + 4:]
        sid = lax.axis_index("s")
        wid = sid * nc + lax.axis_index("c")
        bidx = wid // (SEQ // b_per_w)
        seq_base = (wid % (SEQ // b_per_w)) * b_per_w
        pltpu.sync_copy(pos_hbm.at[bidx, pl.ds(seq_base, b_per_w)], idx_v)

        def clamp_body(i, carry):
            off = pl.multiple_of(i * 16, 16)
            idx_v[pl.ds(off, 16)] = jnp.clip(idx_v[pl.ds(off, 16)], 0, V - 1)
            return carry

        lax.fori_loop(0, b_per_w // 16, clamp_body, 0)

        def start(chunk, buf):
            off = pl.multiple_of(chunk * _CHUNK, _CHUNK)
            pltpu.async_copy(
                table_hbm.at[idx_v.at[pl.ds(off, _CHUNK)]], rows_v.at[buf], gsem[buf]
            )

        def wait_gather(buf):
            pltpu.make_async_copy(
                table_hbm.at[idx_v.at[pl.ds(0, _CHUNK)]], rows_v.at[buf], gsem[buf]
            ).wait()

        def out_slice(chunk):
            off = pl.multiple_of(seq_base + chunk * _CHUNK, _CHUNK)
            return out_hbm.at[bidx, pl.ds(off, _CHUNK)]

        def slot_ref(s):
            return stage_v.at[sid, s]

        for b in range(_OG):
            start(b, b)

        def body8(g, carry):
            for k in range(8):
                chunk = 8 * g + k
                b = k % _NBUF
                nxt = chunk + _OG
                bn = (b + _OG) % _NBUF

                @pl.when(nxt < n_chunks)
                def _():
                    @pl.when(nxt >= _NBUF)
                    def _():
                        if bn == 0:
                            pltpu.make_async_copy(
                                rows_v.at[bn], out_slice(0), wsem[0]
                            ).wait()
                        else:
                            bnp = bn - 1
                            pltpu.make_async_copy(
                                rows_v.at[bn], slot_ref(0), h1sem[bnp]
                            ).wait()
                            pltpu.async_copy(
                                slot_ref(bnp), out_slice(nxt - _NBUF), h2sem[bnp]
                            )

                    start(nxt, bn)

                wait_gather(b)
                if b == 0:
                    pltpu.async_copy(rows_v.at[b], out_slice(chunk), wsem[0])
                else:
                    bp = b - 1

                    @pl.when(chunk >= _NBUF + 1)
                    def _():
                        pltpu.make_async_copy(
                            slot_ref(bp), out_slice(0), h2sem[bp]
                        ).wait()

                    pltpu.async_copy(rows_v.at[b], slot_ref(bp), h1sem[bp])
            return carry

        lax.fori_loop(0, n_chunks // 8, body8, 0)

        for bp, last in ((0, n_chunks - 3), (1, n_chunks - 2), (2, n_chunks - 1)):
            pltpu.make_async_copy(rows_v.at[bp + 1], slot_ref(0), h1sem[bp]).wait()
            pltpu.async_copy(slot_ref(bp), out_slice(last), h2sem[bp])
        for s in range(3):
            pltpu.make_async_copy(slot_ref(s), out_slice(0), h2sem[s]).wait()
        pltpu.make_async_copy(rows_v.at[0], out_slice(0), wsem[0]).wait()

    return gather_kernel


def kernel(positions, table):
    return _make_gather(
        positions.shape[0], positions.shape[1], table.shape[0], table.shape[1]
    )(positions, table)
